# baseline (speedup 1.0000x reference)
import jax
import jax.numpy as jnp
from jax.experimental import pallas as pl

BATCH = 128
MEM = 4096
VAL = 64
B_BLK = 8
M_BLK = 2048


def _update_kernel(mem_ref, w_ref, v_ref, out_ref):
    out_ref[...] = (
        mem_ref[...]
        + w_ref[...][:, :, None] * v_ref[...][:, None, :]
    )


def kernel(memory, w, v):
    grid = (BATCH // B_BLK, MEM // M_BLK)
    return pl.pallas_call(
        _update_kernel,
        grid=grid,
        in_specs=[
            pl.BlockSpec((B_BLK, M_BLK, VAL), lambda i, j: (i, j, 0)),
            pl.BlockSpec((B_BLK, M_BLK), lambda i, j: (i, j)),
            pl.BlockSpec((B_BLK, VAL), lambda i, j: (i, 0)),
        ],
        out_specs=pl.BlockSpec((B_BLK, M_BLK, VAL), lambda i, j: (i, j, 0)),
        out_shape=jax.ShapeDtypeStruct((BATCH, MEM, VAL), memory.dtype),
    )(memory, w, v)
